# Initial kernel scaffold; baseline (speedup 1.0000x reference)
import jax
import jax.numpy as jnp
from jax.experimental import pallas as pl


def kernel(indices, table):
    raise NotImplementedError("write your pallas kernel here")


# speedup vs baseline: 4.8073x; 4.8073x over previous
import functools

import jax
import jax.numpy as jnp
from jax import lax
from jax.experimental import pallas as pl
from jax.experimental.pallas import tpu as pltpu
from jax.experimental.pallas import tpu_sc as plsc

NC = 2
NS = 16
NW = NC * NS

D = 32
C = 128
K = 8


def _gather(table, idx):
    ngrp = idx.shape[1]
    mesh = plsc.VectorSubcoreMesh(core_axis_name="c", subcore_axis_name="s")

    @functools.partial(
        pl.kernel,
        mesh=mesh,
        out_type=jax.ShapeDtypeStruct((NW, ngrp, K, C, D), jnp.float32),
        scratch_types=[
            pltpu.VMEM((K, C), jnp.int32),
            pltpu.VMEM((K, C, D), jnp.float32),
            pltpu.SemaphoreType.DMA,
        ],
        compiler_params=pltpu.CompilerParams(use_tc_tiling_on_sc=False),
    )
    def k(table_hbm, idx_hbm, out_hbm, idx_v, rows_v, sem):
        wid = lax.axis_index("s") * NC + lax.axis_index("c")

        def grp(g, carry):
            pltpu.sync_copy(idx_hbm.at[wid, g], idx_v)
            handles = [
                pltpu.async_copy(table_hbm.at[idx_v.at[j]], rows_v.at[j], sem)
                for j in range(K)
            ]
            for h in handles:
                h.wait()
            pltpu.sync_copy(rows_v, out_hbm.at[wid, g])
            return carry

        lax.fori_loop(0, ngrp, grp, 0)

    return k(table, idx)


def kernel(indices, table):
    B, H = indices.shape
    total = B * H
    per_w = total // NW
    ngrp = per_w // (K * C)
    idx = indices.reshape(NW, ngrp, K, C).astype(jnp.int32)
    out = _gather(table, idx)
    return out.reshape(B, H, D)


# speedup vs baseline: 5.0523x; 1.0510x over previous
import functools

import jax
import jax.numpy as jnp
from jax import lax
from jax.experimental import pallas as pl
from jax.experimental.pallas import tpu as pltpu
from jax.experimental.pallas import tpu_sc as plsc

NC = 2
NS = 16
NW = NC * NS

D = 32
C = 128
K = 10


def _gather(table, idx):
    ngrp = idx.shape[1]
    assert ngrp % 2 == 0
    mesh = plsc.VectorSubcoreMesh(core_axis_name="c", subcore_axis_name="s")

    @functools.partial(
        pl.kernel,
        mesh=mesh,
        out_type=jax.ShapeDtypeStruct((NW, ngrp, K, C, D), jnp.float32),
        scratch_types=[
            pltpu.VMEM((2, K, C), jnp.int32),
            pltpu.VMEM((2, K, C, D), jnp.float32),
            [pltpu.SemaphoreType.DMA] * 2,
            [pltpu.SemaphoreType.DMA] * 2,
            [pltpu.SemaphoreType.DMA] * 2,
        ],
        compiler_params=pltpu.CompilerParams(use_tc_tiling_on_sc=False),
    )
    def k(table_hbm, idx_hbm, out_hbm, idx_v, rows_v, isems, gsems, osems):
        wid = lax.axis_index("s") * NC + lax.axis_index("c")

        def step(g, p, q):
            pltpu.make_async_copy(idx_hbm.at[wid, g], idx_v.at[p], isems[p]).wait()

            @pl.when(g >= 2)
            def _():
                pltpu.make_async_copy(rows_v.at[p], out_hbm.at[wid, 0], osems[p]).wait()

            for j in range(K):
                pltpu.async_copy(table_hbm.at[idx_v.at[p, j]], rows_v.at[p, j], gsems[p])

            @pl.when(g >= 1)
            def _():
                pltpu.make_async_copy(out_hbm.at[wid, 0], rows_v.at[q], gsems[q]).wait()
                pltpu.async_copy(rows_v.at[q], out_hbm.at[wid, g - 1], osems[q])

            @pl.when(g + 1 < ngrp)
            def _():
                pltpu.async_copy(idx_hbm.at[wid, g + 1], idx_v.at[q], isems[q])

        pltpu.async_copy(idx_hbm.at[wid, 0], idx_v.at[0], isems[0])

        def grp2(g2, carry):
            step(2 * g2, 0, 1)
            step(2 * g2 + 1, 1, 0)
            return carry

        lax.fori_loop(0, ngrp // 2, grp2, 0)

        pltpu.make_async_copy(out_hbm.at[wid, 0], rows_v.at[1], gsems[1]).wait()
        pltpu.async_copy(rows_v.at[1], out_hbm.at[wid, ngrp - 1], osems[1])
        pltpu.make_async_copy(rows_v.at[0], out_hbm.at[wid, 0], osems[0]).wait()
        pltpu.make_async_copy(rows_v.at[1], out_hbm.at[wid, 0], osems[1]).wait()

    return k(table, idx)


def kernel(indices, table):
    B, H = indices.shape
    total = B * H
    per_w = total // NW
    ngrp = per_w // (K * C)
    idx = indices.reshape(NW, ngrp, K, C).astype(jnp.int32)
    out = _gather(table, idx)
    return out.reshape(B, H, D)
